# baseline (speedup 1.0000x reference)
import functools

import jax
import jax.numpy as jnp
from jax import lax
from jax.experimental import pallas as pl
from jax.experimental.pallas import tpu as pltpu
from jax.experimental.pallas import tpu_sc as plsc

_B = 4
_T = 2048
_D = 768
_NC = 2
_NS = 16
_NW = _NC * _NS
_PPW = _T // _NW
_CH = 16
_SUB = _PPW // _CH
_NCH = _B * _SUB
_NV = _D // 16
_NBUF = 6
_DEPTH = 4


def _emb_kernel(idx_hbm, tok_hbm, pos_hbm, out_hbm,
                idx_v, pos_v, bufs, isem, psem, gsems, osems):
    wid = lax.axis_index("s") * _NC + lax.axis_index("c")
    pos_base = wid * _PPW

    pos_d = pltpu.async_copy(pos_hbm.at[pl.ds(pos_base, _PPW)], pos_v, psem)
    idx_d = [
        pltpu.async_copy(
            idx_hbm.at[b, pl.ds(pos_base, _PPW)], idx_v.at[b], isem
        )
        for b in range(_B)
    ]

    def add_rows(buf, s):
        def row_body(j, _):
            p = s * _CH + j
            for k in range(_NV):
                col = k * 16
                plsc.addupdate(
                    buf.at[j, pl.ds(col, 16)], pos_v[p, pl.ds(col, 16)]
                )
            return ()

        lax.fori_loop(0, _CH, row_body, ())

    def start_gather(c):
        b, s = c // _SUB, c % _SUB
        if c % _SUB == 0:
            idx_d[b].wait()
        return pltpu.async_copy(
            tok_hbm.at[idx_v.at[b, pl.ds(s * _CH, _CH)]],
            bufs[c % _NBUF],
            gsems[c % _NBUF],
        )

    gd = [None] * _NCH
    od = [None] * _NCH
    for c in range(_DEPTH):
        gd[c] = start_gather(c)
    pos_d.wait()
    for c in range(_NCH):
        p = c % _NBUF
        b, s = c // _SUB, c % _SUB
        gd[c].wait()
        if c + _DEPTH < _NCH:
            if c - 2 >= 0:
                od[c - 2].wait()
            gd[c + _DEPTH] = start_gather(c + _DEPTH)
        add_rows(bufs[p], s)
        od[c] = pltpu.async_copy(
            bufs[p],
            out_hbm.at[b, pl.ds(pos_base + s * _CH, _CH)],
            osems[p],
        )
    for c in range(_NCH - _DEPTH - 2, _NCH):
        od[c].wait()


def kernel(idx, tok_weight, pos_weight):
    idx32 = idx.astype(jnp.int32)
    mesh = plsc.VectorSubcoreMesh(core_axis_name="c", subcore_axis_name="s")
    run = functools.partial(
        pl.kernel,
        out_type=jax.ShapeDtypeStruct((_B, _T, _D), jnp.float32),
        mesh=mesh,
        scratch_types=[
            pltpu.VMEM((_B, _PPW), jnp.int32),
            pltpu.VMEM((_PPW, _D), jnp.float32),
            [pltpu.VMEM((_CH, _D), jnp.float32) for _ in range(_NBUF)],
            pltpu.SemaphoreType.DMA,
            pltpu.SemaphoreType.DMA,
            [pltpu.SemaphoreType.DMA for _ in range(_NBUF)],
            [pltpu.SemaphoreType.DMA for _ in range(_NBUF)],
        ],
    )(_emb_kernel)
    return run(idx32, tok_weight, pos_weight)


# speedup vs baseline: 1.1054x; 1.1054x over previous
import functools

import jax
import jax.numpy as jnp
from jax import lax
from jax.experimental import pallas as pl
from jax.experimental.pallas import tpu as pltpu
from jax.experimental.pallas import tpu_sc as plsc

_B = 4
_T = 2048
_D = 768
_NC = 2
_NS = 16
_NW = _NC * _NS
_PPW = _T // _NW
_CH = 32
_SUB = _PPW // _CH
_NCH = _B * _SUB
_NV = _D // 16
_NBUF = 3
_DEPTH = 2


def _emb_kernel(idx_hbm, tok_hbm, pos_hbm, out_hbm,
                idx_v, pos_v, bufs, isem, psem, gsems, osems):
    wid = lax.axis_index("s") * _NC + lax.axis_index("c")
    pos_base = wid * _PPW

    pos_d = pltpu.async_copy(pos_hbm.at[pl.ds(pos_base, _PPW)], pos_v, psem)
    idx_d = [
        pltpu.async_copy(
            idx_hbm.at[b, pl.ds(pos_base, _PPW)], idx_v.at[b], isem
        )
        for b in range(_B)
    ]

    def add_rows(buf, s):
        @plsc.parallel_loop(0, _CH, 1, unroll=2)
        def row_body(j):
            p = s * _CH + j
            for k in range(_NV):
                col = k * 16
                plsc.addupdate(
                    buf.at[j, pl.ds(col, 16)], pos_v[p, pl.ds(col, 16)]
                )

    def start_gather(c):
        b, s = c // _SUB, c % _SUB
        if s == 0:
            idx_d[b].wait()
        return pltpu.async_copy(
            tok_hbm.at[idx_v.at[b, pl.ds(s * _CH, _CH)]],
            bufs[c % _NBUF],
            gsems[c % _NBUF],
        )

    gd = [None] * _NCH
    od = [None] * _NCH
    for c in range(_DEPTH):
        gd[c] = start_gather(c)
    pos_d.wait()
    for c in range(_NCH):
        p = c % _NBUF
        b, s = c // _SUB, c % _SUB
        gd[c].wait()
        if c + _DEPTH < _NCH:
            prev = c + _DEPTH - _NBUF
            if prev >= 0:
                od[prev].wait()
            gd[c + _DEPTH] = start_gather(c + _DEPTH)
        add_rows(bufs[p], s)
        od[c] = pltpu.async_copy(
            bufs[p],
            out_hbm.at[b, pl.ds(pos_base + s * _CH, _CH)],
            osems[p],
        )
    for c in range(max(0, _NCH - _NBUF), _NCH):
        od[c].wait()


def kernel(idx, tok_weight, pos_weight):
    idx32 = idx.astype(jnp.int32)
    mesh = plsc.VectorSubcoreMesh(core_axis_name="c", subcore_axis_name="s")
    run = functools.partial(
        pl.kernel,
        out_type=jax.ShapeDtypeStruct((_B, _T, _D), jnp.float32),
        mesh=mesh,
        scratch_types=[
            pltpu.VMEM((_B, _PPW), jnp.int32),
            pltpu.VMEM((_PPW, _D), jnp.float32),
            [pltpu.VMEM((_CH, _D), jnp.float32) for _ in range(_NBUF)],
            pltpu.SemaphoreType.DMA,
            pltpu.SemaphoreType.DMA,
            [pltpu.SemaphoreType.DMA for _ in range(_NBUF)],
            [pltpu.SemaphoreType.DMA for _ in range(_NBUF)],
        ],
    )(_emb_kernel)
    return run(idx32, tok_weight, pos_weight)
